# Initial kernel scaffold; baseline (speedup 1.0000x reference)
import jax
import jax.numpy as jnp
from jax.experimental import pallas as pl


def kernel(x, edge_index, W, b):
    raise NotImplementedError("write your pallas kernel here")


# speedup vs baseline: 9.7805x; 9.7805x over previous
import functools

import jax
import jax.numpy as jnp
import numpy as np
from jax import lax
from jax.experimental import pallas as pl
from jax.experimental.pallas import tpu as pltpu
from jax.experimental.pallas import tpu_sc as plsc

N_NODES = 10000
N_PAD = 10240
N_EDGES = 160000
C_IN = 256
C_OUT = 256
HALF = 128

NC = 2
NS = 16
NW = NC * NS

DEG_E = N_EDGES // NW
DEG_EP = 5008
TILE_E = N_EDGES // NS
TILE_EP = 10240
CHUNK = 128
NCHUNK = TILE_EP // CHUNK
ROWS_PER_TILE = N_PAD // NS


def _i32(v):
    return jnp.asarray(v, jnp.int32)


_IDX0 = np.int32(0)


def _deg_body(tar_ref, deg_out_ref, deg_local, tar_v):
    c = lax.axis_index("c")
    s = lax.axis_index("s")
    wid = c * _i32(NS) + s

    def zero(i, carry):
        deg_local[pl.ds(i * _i32(16), 16)] = jnp.zeros((16,), jnp.float32)
        return carry

    lax.fori_loop(_i32(0), _i32(N_PAD // 16), zero, _i32(0))

    pltpu.sync_copy(tar_ref.at[wid], tar_v)
    ones = jnp.ones((16,), jnp.float32)

    def accum(i, carry):
        idx = tar_v[pl.ds(i * _i32(16), 16)]
        plsc.addupdate_scatter(deg_local, [idx], ones)
        return carry

    lax.fori_loop(_i32(0), _i32(DEG_EP // 16), accum, _i32(0))
    pltpu.sync_copy(deg_local, deg_out_ref.at[wid])


_deg_call = pl.kernel(
    _deg_body,
    out_type=jax.ShapeDtypeStruct((NW, N_PAD), jnp.float32),
    mesh=plsc.VectorSubcoreMesh(core_axis_name="c", subcore_axis_name="s", num_cores=NC, num_subcores=NS),
    compiler_params=pltpu.CompilerParams(needs_layout_passes=False),
    scratch_types=[
        pltpu.VMEM((N_PAD,), jnp.float32),
        pltpu.VMEM((DEG_EP,), jnp.int32),
    ],
)


def _prep_body(x_ref, w_ref, b_ref, degp_ref, g2_ref):
    h = lax.dot_general(x_ref[...], w_ref[...],
                        (((1,), (1,)), ((), ())),
                        preferred_element_type=jnp.float32)
    h = h + b_ref[...]
    deg = jnp.sum(degp_ref[...], axis=0) + 1.0
    d = lax.rsqrt(deg)
    g = h * d[:, None]
    g2_ref[0] = g[:, :HALF]
    g2_ref[1] = g[:, HALF:]


def _prep_call(x_pad, w, b2, degp):
    blk = 512
    return pl.pallas_call(
        _prep_body,
        grid=(N_PAD // blk,),
        in_specs=[
            pl.BlockSpec((blk, C_IN), lambda i: (i, _IDX0)),
            pl.BlockSpec((C_OUT, C_IN), lambda i: (_IDX0, _IDX0)),
            pl.BlockSpec((1, C_OUT), lambda i: (_IDX0, _IDX0)),
            pl.BlockSpec((NW, blk), lambda i: (_IDX0, i)),
        ],
        out_specs=pl.BlockSpec((2, blk, HALF), lambda i: (_IDX0, i, _IDX0)),
        out_shape=jax.ShapeDtypeStruct((2, N_PAD, HALF), jnp.float32),
    )(x_pad, w, b2, degp)


def _edge_body(g2f_ref, src_ref, tar_ref, out_ref,
               acc, src_v, tar_v, buf, sem):
    c = lax.axis_index("c")
    s = lax.axis_index("s")
    wid = c * _i32(NS) + s

    pltpu.sync_copy(g2f_ref.at[pl.ds(c * _i32(N_PAD) + s * _i32(ROWS_PER_TILE),
                                     ROWS_PER_TILE)],
                    acc.at[pl.ds(s * _i32(ROWS_PER_TILE), ROWS_PER_TILE)])
    pltpu.sync_copy(src_ref.at[wid], src_v)
    pltpu.sync_copy(tar_ref.at[wid], tar_v)
    plsc.subcore_barrier()

    def step(j, carry):
        pltpu.async_copy(g2f_ref.at[src_v.at[j]], buf, sem).wait()
        pltpu.sync_copy(buf, acc.at[tar_v.at[j]], add=True)
        return carry

    lax.fori_loop(_i32(0), _i32(NCHUNK), step, _i32(0))
    plsc.subcore_barrier()

    pltpu.sync_copy(acc.at[pl.ds(s * _i32(ROWS_PER_TILE), ROWS_PER_TILE)],
                    out_ref.at[pl.ds(c * _i32(N_PAD) + s * _i32(ROWS_PER_TILE),
                                     ROWS_PER_TILE)])


_edge_call = pl.kernel(
    _edge_body,
    out_type=jax.ShapeDtypeStruct((2 * N_PAD, HALF), jnp.float32),
    mesh=plsc.VectorSubcoreMesh(core_axis_name="c", subcore_axis_name="s", num_cores=NC, num_subcores=NS),
    scratch_types=[
        pltpu.VMEM_SHARED((N_PAD, HALF), jnp.float32),
        pltpu.VMEM((NCHUNK, CHUNK), jnp.int32),
        pltpu.VMEM((NCHUNK, CHUNK), jnp.int32),
        pltpu.VMEM((CHUNK, HALF), jnp.float32),
        pltpu.SemaphoreType.DMA,
    ],
)


def _comb_body(acc2_ref, degt_ref, out_ref):
    deg = jnp.sum(degt_ref[...], axis=1) + 1.0
    d = lax.rsqrt(deg)[:, None]
    out_ref[:, :HALF] = acc2_ref[0] * d
    out_ref[:, HALF:] = acc2_ref[1] * d


def _comb_call(acc2, degt):
    blk = 400
    return pl.pallas_call(
        _comb_body,
        grid=(N_NODES // blk,),
        in_specs=[
            pl.BlockSpec((2, blk, HALF), lambda i: (_IDX0, i, _IDX0)),
            pl.BlockSpec((blk, NW), lambda i: (i, _IDX0)),
        ],
        out_specs=pl.BlockSpec((blk, C_OUT), lambda i: (i, _IDX0)),
        out_shape=jax.ShapeDtypeStruct((N_NODES, C_OUT), jnp.float32),
    )(acc2, degt)


def kernel(x, edge_index, W, b):
    src = edge_index[1].astype(jnp.int32)
    tar = edge_index[0].astype(jnp.int32)

    tar_pad = jnp.concatenate(
        [tar.reshape(NW, DEG_E),
         jnp.full((NW, DEG_EP - DEG_E), N_NODES, jnp.int32)], axis=1)
    degp = _deg_call(tar_pad)

    x_pad = jnp.pad(x, ((0, N_PAD - N_NODES), (0, 0)))
    g2 = _prep_call(x_pad, W, b.reshape(1, C_OUT), degp)

    g2f = g2.reshape(2 * N_PAD, HALF)
    src_p = jnp.pad(src.reshape(NS, TILE_E), ((0, 0), (0, TILE_EP - TILE_E)))
    src_p = src_p.reshape(NS, NCHUNK, CHUNK)
    src_sc = jnp.concatenate([src_p, src_p + N_PAD], axis=0)
    tar_p = jnp.pad(tar.reshape(NS, TILE_E), ((0, 0), (0, TILE_EP - TILE_E)),
                    constant_values=N_NODES)
    tar_p = tar_p.reshape(NS, NCHUNK, CHUNK)
    tar_sc = jnp.concatenate([tar_p, tar_p], axis=0)
    accf = _edge_call(g2f, src_sc, tar_sc)

    acc2 = accf.reshape(2, N_PAD, HALF)
    return _comb_call(acc2, degp.T)
